# baseline (speedup 1.0000x reference)
import functools

import jax
import jax.numpy as jnp
from jax import lax
from jax.experimental import pallas as pl
from jax.experimental.pallas import tpu as pltpu
from jax.experimental.pallas import tpu_sc as plsc

N_VOCAB = 49408
N_EMBD = 768
N_TOKEN = 77
BATCH = 4096

NC = 2
NS = 16
NW = NC * NS
LANES = 16
SLICES = N_EMBD // LANES

BPT = BATCH // NW
KB = 16
SUBS = BPT // KB
NBUF = 4

_mesh = plsc.VectorSubcoreMesh(core_axis_name="c", subcore_axis_name="s")


@functools.partial(
    pl.kernel,
    out_type=jax.ShapeDtypeStruct((N_TOKEN, BATCH, N_EMBD), jnp.float32),
    mesh=_mesh,
    scratch_types=[
        pltpu.VMEM((N_TOKEN, N_EMBD), jnp.float32),
        pltpu.VMEM((N_TOKEN, BPT), jnp.int32),
        pltpu.VMEM((NBUF, KB, N_EMBD), jnp.float32),
        pltpu.SemaphoreType.DMA((NBUF,)),
        pltpu.SemaphoreType.DMA((NBUF,)),
    ],
)
def _embed_kernel(tokens_hbm, table_hbm, pos_hbm, out_hbm,
                  pos_v, idx_v, rows_v, gsem, ssem):
    wid = lax.axis_index("s") * NC + lax.axis_index("c")
    b0 = wid * BPT

    pltpu.sync_copy(pos_hbm, pos_v)
    pltpu.sync_copy(tokens_hbm.at[wid], idx_v)

    def gather_desc(s, sub):
        return pltpu.make_async_copy(
            table_hbm.at[idx_v.at[s, pl.ds(sub * KB, KB)]],
            rows_v.at[sub % NBUF], gsem.at[sub % NBUF])

    def scatter_desc(s, sub):
        off = pl.multiple_of(b0 + sub * KB, KB)
        return pltpu.make_async_copy(
            rows_v.at[sub % NBUF], out_hbm.at[s, pl.ds(off, KB), :],
            ssem.at[sub % NBUF])

    def step(s, sub):
        slot = sub % NBUF
        gather_desc(s, sub).wait()

        for sl in range(SLICES):
            pv = pos_v[s, pl.ds(sl * LANES, LANES)]
            for r in range(KB):
                plsc.addupdate(rows_v.at[slot, r, pl.ds(sl * LANES, LANES)], pv)

        scatter_desc(s, sub).start()

        if sub >= 1:
            scatter_desc(s, sub - 1).wait()
        else:
            @pl.when(s >= 1)
            def _():
                scatter_desc(s - 1, SUBS - 1).wait()

        if sub < SUBS - NBUF + 1:
            gather_desc(s, sub + NBUF - 1).start()
        else:
            @pl.when(s + 1 < N_TOKEN)
            def _():
                gather_desc(s + 1, sub - SUBS + NBUF - 1).start()

    for sub in range(NBUF - 1):
        gather_desc(0, sub).start()

    @pl.loop(0, N_TOKEN)
    def _ring(s):
        for sub in range(SUBS):
            step(s, sub)

    scatter_desc(N_TOKEN - 1, SUBS - 1).wait()


def kernel(tokens, table, pos):
    ids = tokens.astype(jnp.int32).T.reshape(N_TOKEN, NW, BPT).transpose(1, 0, 2)
    out = _embed_kernel(ids, table, pos)
    return out.transpose(1, 0, 2)
